# Initial kernel scaffold; baseline (speedup 1.0000x reference)
import jax
import jax.numpy as jnp
from jax.experimental import pallas as pl


def kernel(x, tables):
    raise NotImplementedError("write your pallas kernel here")


# speedup vs baseline: 2.9993x; 2.9993x over previous
import functools
import math

import jax
import jax.numpy as jnp
from jax import lax
from jax.experimental import pallas as pl
from jax.experimental.pallas import tpu as pltpu
from jax.experimental.pallas import tpu_sc as plsc

IN_DIM = 3
N_LEVELS = 16
F = 2
LOG2_HASHMAP = 19
BASE_RES = 16
DESIRED_RES = 512
N_POINTS = 524288

P1 = 2654435761
P2 = 805459861

_beta = math.exp((math.log(DESIRED_RES) - math.log(BASE_RES)) / (BASE_RES - 1))
_RES = [int(math.floor(BASE_RES * _beta ** l)) for l in range(N_LEVELS)]
_HS = [min(r ** IN_DIM, 2 ** LOG2_HASHMAP) for r in _RES]
_OFF = [0]
for _h in _HS:
    _OFF.append(_OFF[-1] + _h)
TOTAL_ROWS = _OFF[-1]

NC = 2
NS = 16
NW = NC * NS
LANES = 16

PTS_PER_W = N_POINTS // NW
CP = 512
N_CHUNKS = PTS_PER_W // CP
NG = CP // LANES

N_STAGED = 3
STAGE_ROWS = _OFF[N_STAGED]
STAGE_PAD = (STAGE_ROWS + 7) // 8 * 8

GSZ = 128
NGATH = 8 * CP // GSZ


def _hash_mod(h_u32, hs):
    if hs & (hs - 1) == 0:
        return jnp.bitwise_and(h_u32, jnp.uint32(hs - 1))
    return jnp.remainder(h_u32, jnp.uint32(hs))


def _corner_setup(xv, b, res):
    x0 = xv[0, pl.ds(b, LANES)]
    x1 = xv[1, pl.ds(b, LANES)]
    x2 = xv[2, pl.ds(b, LANES)]
    r = jnp.float32(res)
    xs0, xs1, xs2 = x0 * r, x1 * r, x2 * r
    xi0 = xs0.astype(jnp.int32)
    xi1 = xs1.astype(jnp.int32)
    xi2 = xs2.astype(jnp.int32)
    xf0 = xs0 - xi0.astype(jnp.float32)
    xf1 = xs1 - xi1.astype(jnp.float32)
    xf2 = xs2 - xi2.astype(jnp.float32)
    a0 = xi0.astype(jnp.uint32)
    a1 = a0 + jnp.uint32(1)
    b0 = xi1.astype(jnp.uint32) * jnp.uint32(P1)
    b1 = b0 + jnp.uint32(P1)
    c0 = xi2.astype(jnp.uint32) * jnp.uint32(P2)
    c1 = c0 + jnp.uint32(P2)
    u = (jnp.float32(1.0) - xf0, xf0)
    v = (jnp.float32(1.0) - xf1, xf1)
    t = (jnp.float32(1.0) - xf2, xf2)
    return (a0, a1), (b0, b1), (c0, c1), u, v, t


def kernel(x, tables):
    x_t = x.T

    mesh = plsc.VectorSubcoreMesh(core_axis_name="c", subcore_axis_name="s")

    @functools.partial(
        pl.kernel,
        mesh=mesh,
        compiler_params=pltpu.CompilerParams(
            needs_layout_passes=False, use_tc_tiling_on_sc=False),
        out_type=jax.ShapeDtypeStruct((N_POINTS, 2 * N_LEVELS), jnp.float32),
        scratch_types=[
            pltpu.VMEM((IN_DIM, CP), jnp.float32),
            pltpu.VMEM((STAGE_PAD * F,), jnp.float32),
            pltpu.VMEM((8 * CP,), jnp.int32),
            pltpu.VMEM((8 * CP, F), jnp.float32),
            pltpu.VMEM((8 * CP,), jnp.float32),
            pltpu.VMEM((CP, 2 * N_LEVELS), jnp.float32),
            pltpu.SemaphoreType.DMA,
        ],
    )
    def grid_kernel(x_hbm, t_hbm, tflat_hbm, out_hbm, xv, tstage, idxbuf, gbuf,
                    wbuf, obuf, sem):
        wid = lax.axis_index("s") * NC + lax.axis_index("c")

        pltpu.sync_copy(tflat_hbm.at[pl.ds(0, STAGE_PAD * F)], tstage)

        iota = lax.iota(jnp.int32, LANES)
        col0 = jnp.zeros((LANES,), jnp.int32)
        col1 = jnp.ones((LANES,), jnp.int32)

        def chunk_body(ci, carry):
            base = wid * PTS_PER_W + ci * CP
            pltpu.sync_copy(x_hbm.at[:, pl.ds(base, CP)], xv)

            for l in range(N_LEVELS):
                res, hs, off = _RES[l], _HS[l], _OFF[l]

                if l < N_STAGED:
                    def fgroup(g, c, res=res, hs=hs, off=off, lvl=l):
                        b = g * LANES
                        (a0, a1), (b0, b1), (c0, c1), u, v, t = \
                            _corner_setup(xv, b, res)
                        acc0 = jnp.zeros((LANES,), jnp.float32)
                        acc1 = jnp.zeros((LANES,), jnp.float32)
                        for j in range(8):
                            j0, j1, j2 = j & 1, (j >> 1) & 1, (j >> 2) & 1
                            h = (a1 if j0 else a0) ^ (b1 if j1 else b0) \
                                ^ (c1 if j2 else c0)
                            rid = (_hash_mod(h, hs)).astype(jnp.int32) + off
                            w = (u[j0] * v[j1]) * t[j2]
                            fid = rid * 2
                            g0 = plsc.load_gather(tstage, [fid])
                            g1 = plsc.load_gather(tstage, [fid + 1])
                            acc0 = acc0 + g0 * w
                            acc1 = acc1 + g1 * w
                        rows = b + iota
                        plsc.store_scatter(
                            obuf, [rows, jnp.full((LANES,), 2 * lvl, jnp.int32)],
                            acc0)
                        plsc.store_scatter(
                            obuf,
                            [rows, jnp.full((LANES,), 2 * lvl + 1, jnp.int32)],
                            acc1)
                        return c

                    lax.fori_loop(0, NG, fgroup, 0)
                else:
                    def igroup(g, c, res=res, hs=hs, off=off):
                        b = g * LANES
                        (a0, a1), (b0, b1), (c0, c1), u, v, t = \
                            _corner_setup(xv, b, res)
                        for j in range(8):
                            j0, j1, j2 = j & 1, (j >> 1) & 1, (j >> 2) & 1
                            h = (a1 if j0 else a0) ^ (b1 if j1 else b0) \
                                ^ (c1 if j2 else c0)
                            rid = (_hash_mod(h, hs)).astype(jnp.int32) + off
                            w = (u[j0] * v[j1]) * t[j2]
                            idxbuf[pl.ds(j * CP + b, LANES)] = rid
                            wbuf[pl.ds(j * CP + b, LANES)] = w
                        return c

                    lax.fori_loop(0, NG, igroup, 0)

                    copies = [
                        pltpu.async_copy(
                            t_hbm.at[idxbuf.at[pl.ds(gg * GSZ, GSZ)]],
                            gbuf.at[pl.ds(gg * GSZ, GSZ)], sem)
                        for gg in range(NGATH)
                    ]
                    for cpy in copies:
                        cpy.wait()

                    def rgroup(g, c, lvl=l):
                        b = g * LANES
                        acc0 = jnp.zeros((LANES,), jnp.float32)
                        acc1 = jnp.zeros((LANES,), jnp.float32)
                        for j in range(8):
                            pos = j * CP + b
                            wv = wbuf[pl.ds(pos, LANES)]
                            ridx = pos + iota
                            g0 = plsc.load_gather(gbuf, [ridx, col0])
                            g1 = plsc.load_gather(gbuf, [ridx, col1])
                            acc0 = acc0 + g0 * wv
                            acc1 = acc1 + g1 * wv
                        rows = b + iota
                        plsc.store_scatter(
                            obuf, [rows, jnp.full((LANES,), 2 * lvl, jnp.int32)],
                            acc0)
                        plsc.store_scatter(
                            obuf,
                            [rows, jnp.full((LANES,), 2 * lvl + 1, jnp.int32)],
                            acc1)
                        return c

                    lax.fori_loop(0, NG, rgroup, 0)

            pltpu.sync_copy(obuf, out_hbm.at[pl.ds(base, CP)])
            return carry

        lax.fori_loop(0, N_CHUNKS, chunk_body, 0)

    return grid_kernel(x_t, tables, tables.reshape(-1))


# speedup vs baseline: 3.6368x; 1.2125x over previous
import functools
import math

import jax
import jax.numpy as jnp
from jax import lax
from jax.experimental import pallas as pl
from jax.experimental.pallas import tpu as pltpu
from jax.experimental.pallas import tpu_sc as plsc

IN_DIM = 3
N_LEVELS = 16
F = 2
LOG2_HASHMAP = 19
BASE_RES = 16
DESIRED_RES = 512
N_POINTS = 524288

P1 = 2654435761
P2 = 805459861

_beta = math.exp((math.log(DESIRED_RES) - math.log(BASE_RES)) / (BASE_RES - 1))
_RES = [int(math.floor(BASE_RES * _beta ** l)) for l in range(N_LEVELS)]
_HS = [min(r ** IN_DIM, 2 ** LOG2_HASHMAP) for r in _RES]
_OFF = [0]
for _h in _HS:
    _OFF.append(_OFF[-1] + _h)
TOTAL_ROWS = _OFF[-1]

NC = 2
NS = 16
NW = NC * NS
LANES = 16

PTS_PER_W = N_POINTS // NW
CP = 256
N_CHUNKS = PTS_PER_W // CP
NG = CP // LANES

N_TILE_STAGED = 2
STAGE_ROWS = _OFF[N_TILE_STAGED]
STAGE_PAD = (STAGE_ROWS + 7) // 8 * 8

SHARED_LEVELS = [2, 3, 4, 5, 6]
SH_BASE = _OFF[SHARED_LEVELS[0]]
SH_ROWS = _OFF[SHARED_LEVELS[-1] + 1] - SH_BASE
SH_WORDS = SH_ROWS * F
SH_ALLOC = (SH_WORDS + 7) // 8 * 8

HBM_LEVELS = list(range(SHARED_LEVELS[-1] + 1, N_LEVELS))
STREAM_LEVELS = SHARED_LEVELS + HBM_LEVELS

GSZ = 128
NGATH = 2 * 8 * CP // GSZ

OUT_W = 2 * N_LEVELS


def _hash_mod(h_u32, hs):
    if hs & (hs - 1) == 0:
        return jnp.bitwise_and(h_u32, jnp.uint32(hs - 1))
    return jnp.remainder(h_u32, jnp.uint32(hs))


def _corner_setup(xv, b, res):
    x0 = xv[0, pl.ds(b, LANES)]
    x1 = xv[1, pl.ds(b, LANES)]
    x2 = xv[2, pl.ds(b, LANES)]
    r = jnp.float32(res)
    xs0, xs1, xs2 = x0 * r, x1 * r, x2 * r
    xi0 = xs0.astype(jnp.int32)
    xi1 = xs1.astype(jnp.int32)
    xi2 = xs2.astype(jnp.int32)
    xf0 = xs0 - xi0.astype(jnp.float32)
    xf1 = xs1 - xi1.astype(jnp.float32)
    xf2 = xs2 - xi2.astype(jnp.float32)
    a0 = xi0.astype(jnp.uint32)
    a1 = a0 + jnp.uint32(1)
    b0 = xi1.astype(jnp.uint32) * jnp.uint32(P1)
    b1 = b0 + jnp.uint32(P1)
    c0 = xi2.astype(jnp.uint32) * jnp.uint32(P2)
    c1 = c0 + jnp.uint32(P2)
    u = (jnp.float32(1.0) - xf0, xf0)
    v = (jnp.float32(1.0) - xf1, xf1)
    t = (jnp.float32(1.0) - xf2, xf2)
    return (a0, a1), (b0, b1), (c0, c1), u, v, t


def kernel(x, tables):
    x_t = x.T
    tflat = tables.reshape(-1)

    mesh = plsc.VectorSubcoreMesh(core_axis_name="c", subcore_axis_name="s")

    @functools.partial(
        pl.kernel,
        mesh=mesh,
        compiler_params=pltpu.CompilerParams(
            needs_layout_passes=False, use_tc_tiling_on_sc=False),
        out_type=jax.ShapeDtypeStruct((N_POINTS * OUT_W,), jnp.float32),
        scratch_types=[
            pltpu.VMEM((IN_DIM, CP), jnp.float32),
            pltpu.VMEM((STAGE_PAD * F,), jnp.float32),
            pltpu.VMEM_SHARED((SH_ALLOC,), jnp.float32),
            pltpu.VMEM((2 * 8 * CP,), jnp.int32),
            pltpu.VMEM((2 * 8 * CP,), jnp.float32),
            pltpu.VMEM((8 * CP,), jnp.float32),
            pltpu.VMEM((CP * OUT_W,), jnp.float32),
            pltpu.SemaphoreType.DMA,
        ],
    )
    def grid_kernel(x_hbm, tflat_hbm, out_hbm, xv, tstage, shstage,
                    idxbuf, gbuf, wbuf, obuf, sem):
        wid = lax.axis_index("s") * NC + lax.axis_index("c")

        pltpu.sync_copy(tflat_hbm.at[pl.ds(0, STAGE_PAD * F)], tstage)

        @pl.when(lax.axis_index("s") == 0)
        def _stage_shared():
            pltpu.sync_copy(tflat_hbm.at[pl.ds(SH_BASE * F, SH_ALLOC)],
                            shstage)

        plsc.subcore_barrier()

        iota = lax.iota(jnp.int32, LANES)
        oidx = iota * OUT_W

        def out_store(b, l, acc0, acc1):
            sidx = oidx + (b * OUT_W + 2 * l)
            plsc.store_scatter(obuf, [sidx], acc0)
            plsc.store_scatter(obuf, [sidx + 1], acc1)

        def run_fused_level(l):
            res, hs, off = _RES[l], _HS[l], _OFF[l]

            def fgroup(g, c):
                b = g * LANES
                (a0, a1), (b0, b1), (c0, c1), u, v, t = \
                    _corner_setup(xv, b, res)
                acc0 = jnp.zeros((LANES,), jnp.float32)
                acc1 = jnp.zeros((LANES,), jnp.float32)
                for j in range(8):
                    j0, j1, j2 = j & 1, (j >> 1) & 1, (j >> 2) & 1
                    h = (a1 if j0 else a0) ^ (b1 if j1 else b0) \
                        ^ (c1 if j2 else c0)
                    rid = (_hash_mod(h, hs)).astype(jnp.int32) + off
                    w = (u[j0] * v[j1]) * t[j2]
                    fid = rid * 2
                    g0 = plsc.load_gather(tstage, [fid])
                    g1 = plsc.load_gather(tstage, [fid + 1])
                    acc0 = acc0 + g0 * w
                    acc1 = acc1 + g1 * w
                out_store(b, l, acc0, acc1)
                return c

            lax.fori_loop(0, NG, fgroup, 0)

        def gen_indices(l):
            res, hs, off = _RES[l], _HS[l], _OFF[l]
            if l in SHARED_LEVELS:
                word_off = (off - SH_BASE) * F
            else:
                word_off = off * F

            def igroup(g, c):
                b = g * LANES
                (a0, a1), (b0, b1), (c0, c1), u, v, t = \
                    _corner_setup(xv, b, res)
                for j in range(8):
                    j0, j1, j2 = j & 1, (j >> 1) & 1, (j >> 2) & 1
                    h = (a1 if j0 else a0) ^ (b1 if j1 else b0) \
                        ^ (c1 if j2 else c0)
                    fid = (_hash_mod(h, hs)).astype(jnp.int32) * 2 + word_off
                    w = (u[j0] * v[j1]) * t[j2]
                    idxbuf[pl.ds(j * CP + b, LANES)] = fid
                    idxbuf[pl.ds(8 * CP + j * CP + b, LANES)] = fid + 1
                    wbuf[pl.ds(j * CP + b, LANES)] = w
                return c

            lax.fori_loop(0, NG, igroup, 0)

        def run_stream_level(l):
            gen_indices(l)
            src = shstage if l in SHARED_LEVELS else tflat_hbm
            copies = [
                pltpu.async_copy(
                    src.at[idxbuf.at[pl.ds(gg * GSZ, GSZ)]],
                    gbuf.at[pl.ds(gg * GSZ, GSZ)], sem)
                for gg in range(NGATH)
            ]
            for cpy in copies:
                cpy.wait()

            def rgroup(g, c):
                b = g * LANES
                acc0 = jnp.zeros((LANES,), jnp.float32)
                acc1 = jnp.zeros((LANES,), jnp.float32)
                for j in range(8):
                    pos = j * CP + b
                    wv = wbuf[pl.ds(pos, LANES)]
                    g0 = gbuf[pl.ds(pos, LANES)]
                    g1 = gbuf[pl.ds(8 * CP + pos, LANES)]
                    acc0 = acc0 + g0 * wv
                    acc1 = acc1 + g1 * wv
                out_store(b, l, acc0, acc1)
                return c

            lax.fori_loop(0, NG, rgroup, 0)

        def chunk_body(ci, carry):
            base = wid * PTS_PER_W + ci * CP
            pltpu.sync_copy(x_hbm.at[:, pl.ds(base, CP)], xv)

            for l in range(N_TILE_STAGED):
                run_fused_level(l)
            for l in STREAM_LEVELS:
                run_stream_level(l)

            pltpu.sync_copy(obuf, out_hbm.at[pl.ds(base * OUT_W, CP * OUT_W)])
            return carry

        lax.fori_loop(0, N_CHUNKS, chunk_body, 0)

    out_flat = grid_kernel(x_t, tflat)
    return out_flat.reshape(N_POINTS, OUT_W)


# speedup vs baseline: 8.9226x; 2.4534x over previous
import functools
import math

import jax
import jax.numpy as jnp
from jax import lax
from jax.experimental import pallas as pl
from jax.experimental.pallas import tpu as pltpu
from jax.experimental.pallas import tpu_sc as plsc

IN_DIM = 3
N_LEVELS = 16
F = 2
LOG2_HASHMAP = 19
BASE_RES = 16
DESIRED_RES = 512
N_POINTS = 524288

P1 = 2654435761
P2 = 805459861

_beta = math.exp((math.log(DESIRED_RES) - math.log(BASE_RES)) / (BASE_RES - 1))
_RES = [int(math.floor(BASE_RES * _beta ** l)) for l in range(N_LEVELS)]
_HS = [min(r ** IN_DIM, 2 ** LOG2_HASHMAP) for r in _RES]
_OFF = [0]
for _h in _HS:
    _OFF.append(_OFF[-1] + _h)
TOTAL_ROWS = _OFF[-1]

NC = 2
NS = 16
NW = NC * NS
LANES = 16

PTS_PER_W = N_POINTS // NW
CP = 256
N_CHUNKS = PTS_PER_W // CP
NG = CP // LANES
GBLK = 8 * LANES

N_TILE_STAGED = 2
STAGE_ROWS = _OFF[N_TILE_STAGED]
STAGE_PAD = (STAGE_ROWS + 7) // 8 * 8

SHARED_LEVELS = [2, 3, 4, 5, 6]
SH_BASE = _OFF[SHARED_LEVELS[0]]
SH_ROWS = _OFF[SHARED_LEVELS[-1] + 1] - SH_BASE
SH_ALLOC = (SH_ROWS + 7) // 8 * 8

HBM_LEVELS = list(range(SHARED_LEVELS[-1] + 1, N_LEVELS))
STREAM_LEVELS = SHARED_LEVELS + HBM_LEVELS

OUT_W = 2 * N_LEVELS


def _hash_mod(h_u32, hs):
    if hs & (hs - 1) == 0:
        return jnp.bitwise_and(h_u32, jnp.uint32(hs - 1))
    return jnp.remainder(h_u32, jnp.uint32(hs))


def _corner_setup(xv, b, res):
    x0 = xv[0, pl.ds(b, LANES)]
    x1 = xv[1, pl.ds(b, LANES)]
    x2 = xv[2, pl.ds(b, LANES)]
    r = jnp.float32(res)
    xs0, xs1, xs2 = x0 * r, x1 * r, x2 * r
    xi0 = xs0.astype(jnp.int32)
    xi1 = xs1.astype(jnp.int32)
    xi2 = xs2.astype(jnp.int32)
    xf0 = xs0 - xi0.astype(jnp.float32)
    xf1 = xs1 - xi1.astype(jnp.float32)
    xf2 = xs2 - xi2.astype(jnp.float32)
    a0 = xi0.astype(jnp.uint32)
    a1 = a0 + jnp.uint32(1)
    b0 = xi1.astype(jnp.uint32) * jnp.uint32(P1)
    b1 = b0 + jnp.uint32(P1)
    c0 = xi2.astype(jnp.uint32) * jnp.uint32(P2)
    c1 = c0 + jnp.uint32(P2)
    u = (jnp.float32(1.0) - xf0, xf0)
    v = (jnp.float32(1.0) - xf1, xf1)
    t = (jnp.float32(1.0) - xf2, xf2)
    return (a0, a1), (b0, b1), (c0, c1), u, v, t


def kernel(x, tables):
    x_t = x.T
    t0 = tables[:, 0]
    t1 = tables[:, 1]

    mesh = plsc.VectorSubcoreMesh(core_axis_name="c", subcore_axis_name="s")

    @functools.partial(
        pl.kernel,
        mesh=mesh,
        compiler_params=pltpu.CompilerParams(
            needs_layout_passes=False, use_tc_tiling_on_sc=False),
        out_type=jax.ShapeDtypeStruct((N_POINTS * OUT_W,), jnp.float32),
        scratch_types=[
            pltpu.VMEM((IN_DIM, CP), jnp.float32),
            pltpu.VMEM((STAGE_PAD,), jnp.float32),
            pltpu.VMEM((STAGE_PAD,), jnp.float32),
            pltpu.VMEM_SHARED((SH_ALLOC,), jnp.float32),
            pltpu.VMEM_SHARED((SH_ALLOC,), jnp.float32),
            pltpu.VMEM((8 * CP,), jnp.int32),
            pltpu.VMEM((2 * 8 * CP,), jnp.float32),
            pltpu.VMEM((8 * CP,), jnp.float32),
            pltpu.VMEM((CP * OUT_W,), jnp.float32),
            pltpu.SemaphoreType.DMA,
        ],
    )
    def grid_kernel(x_hbm, t0_hbm, t1_hbm, out_hbm, xv, ts0, ts1, sh0, sh1,
                    idxbuf, gbuf, wbuf, obuf, sem):
        wid = lax.axis_index("s") * NC + lax.axis_index("c")

        pltpu.sync_copy(t0_hbm.at[pl.ds(0, STAGE_PAD)], ts0)
        pltpu.sync_copy(t1_hbm.at[pl.ds(0, STAGE_PAD)], ts1)

        @pl.when(lax.axis_index("s") == 0)
        def _stage_shared():
            pltpu.sync_copy(t0_hbm.at[pl.ds(SH_BASE, SH_ALLOC)], sh0)
            pltpu.sync_copy(t1_hbm.at[pl.ds(SH_BASE, SH_ALLOC)], sh1)

        plsc.subcore_barrier()

        iota = lax.iota(jnp.int32, LANES)
        oidx = iota * OUT_W

        def out_store(b, l, acc0, acc1):
            sidx = oidx + (b * OUT_W + 2 * l)
            plsc.store_scatter(obuf, [sidx], acc0)
            plsc.store_scatter(obuf, [sidx + 1], acc1)

        def run_fused_level(l):
            res, hs, off = _RES[l], _HS[l], _OFF[l]

            def fgroup(g, c):
                b = g * LANES
                (a0, a1), (b0, b1), (c0, c1), u, v, t = \
                    _corner_setup(xv, b, res)
                acc0 = jnp.zeros((LANES,), jnp.float32)
                acc1 = jnp.zeros((LANES,), jnp.float32)
                for j in range(8):
                    j0, j1, j2 = j & 1, (j >> 1) & 1, (j >> 2) & 1
                    h = (a1 if j0 else a0) ^ (b1 if j1 else b0) \
                        ^ (c1 if j2 else c0)
                    rid = (_hash_mod(h, hs)).astype(jnp.int32) + off
                    w = (u[j0] * v[j1]) * t[j2]
                    g0 = plsc.load_gather(ts0, [rid])
                    g1 = plsc.load_gather(ts1, [rid])
                    acc0 = acc0 + g0 * w
                    acc1 = acc1 + g1 * w
                out_store(b, l, acc0, acc1)
                return c

            lax.fori_loop(0, NG, fgroup, 0)

        def run_stream_level(l):
            res, hs, off = _RES[l], _HS[l], _OFF[l]
            shared = l in SHARED_LEVELS
            row_off = (off - SH_BASE) if shared else off
            s0 = sh0 if shared else t0_hbm
            s1 = sh1 if shared else t1_hbm

            def igroup(g, c):
                b = g * LANES
                blk = g * GBLK
                (a0, a1), (b0, b1), (c0, c1), u, v, t = \
                    _corner_setup(xv, b, res)
                for j in range(8):
                    j0, j1, j2 = j & 1, (j >> 1) & 1, (j >> 2) & 1
                    h = (a1 if j0 else a0) ^ (b1 if j1 else b0) \
                        ^ (c1 if j2 else c0)
                    rid = (_hash_mod(h, hs)).astype(jnp.int32) + row_off
                    w = (u[j0] * v[j1]) * t[j2]
                    idxbuf[pl.ds(blk + j * LANES, LANES)] = rid
                    wbuf[pl.ds(blk + j * LANES, LANES)] = w
                pltpu.async_copy(s0.at[idxbuf.at[pl.ds(blk, GBLK)]],
                                 gbuf.at[pl.ds(blk, GBLK)], sem)
                pltpu.async_copy(s1.at[idxbuf.at[pl.ds(blk, GBLK)]],
                                 gbuf.at[pl.ds(8 * CP + blk, GBLK)], sem)
                return c

            lax.fori_loop(0, NG, igroup, 0)

            for g in range(NG):
                pltpu.make_async_copy(
                    t0_hbm.at[pl.ds(0, GBLK)],
                    gbuf.at[pl.ds(g * GBLK, GBLK)], sem).wait()
                pltpu.make_async_copy(
                    t0_hbm.at[pl.ds(0, GBLK)],
                    gbuf.at[pl.ds(8 * CP + g * GBLK, GBLK)], sem).wait()

            def rgroup(g, c):
                b = g * LANES
                blk = g * GBLK
                acc0 = jnp.zeros((LANES,), jnp.float32)
                acc1 = jnp.zeros((LANES,), jnp.float32)
                for j in range(8):
                    pos = blk + j * LANES
                    wv = wbuf[pl.ds(pos, LANES)]
                    g0 = gbuf[pl.ds(pos, LANES)]
                    g1 = gbuf[pl.ds(8 * CP + pos, LANES)]
                    acc0 = acc0 + g0 * wv
                    acc1 = acc1 + g1 * wv
                out_store(b, l, acc0, acc1)
                return c

            lax.fori_loop(0, NG, rgroup, 0)

        def chunk_body(ci, carry):
            base = wid * PTS_PER_W + ci * CP
            pltpu.sync_copy(x_hbm.at[:, pl.ds(base, CP)], xv)

            run_stream_level(STREAM_LEVELS[0])
            for l in range(N_TILE_STAGED):
                run_fused_level(l)
            for l in STREAM_LEVELS[1:]:
                run_stream_level(l)

            pltpu.sync_copy(obuf, out_hbm.at[pl.ds(base * OUT_W, CP * OUT_W)])
            return carry

        lax.fori_loop(0, N_CHUNKS, chunk_body, 0)

    out_flat = grid_kernel(x_t, t0, t1)
    return out_flat.reshape(N_POINTS, OUT_W)


# speedup vs baseline: 9.4173x; 1.0554x over previous
import functools
import math

import jax
import jax.numpy as jnp
from jax import lax
from jax.experimental import pallas as pl
from jax.experimental.pallas import tpu as pltpu
from jax.experimental.pallas import tpu_sc as plsc

IN_DIM = 3
N_LEVELS = 16
F = 2
LOG2_HASHMAP = 19
BASE_RES = 16
DESIRED_RES = 512
N_POINTS = 524288

P1 = 2654435761
P2 = 805459861

_beta = math.exp((math.log(DESIRED_RES) - math.log(BASE_RES)) / (BASE_RES - 1))
_RES = [int(math.floor(BASE_RES * _beta ** l)) for l in range(N_LEVELS)]
_HS = [min(r ** IN_DIM, 2 ** LOG2_HASHMAP) for r in _RES]
_OFF = [0]
for _h in _HS:
    _OFF.append(_OFF[-1] + _h)
TOTAL_ROWS = _OFF[-1]

NC = 2
NS = 16
NW = NC * NS
LANES = 16

PTS_PER_W = N_POINTS // NW
CP = 512
N_CHUNKS = PTS_PER_W // CP
NG = CP // LANES
GBLK = 8 * LANES

N_TILE_STAGED = 2
STAGE_ROWS = _OFF[N_TILE_STAGED]
STAGE_PAD = (STAGE_ROWS + 7) // 8 * 8

SHARED_LEVELS = [2, 3, 4, 5, 6]
SH_BASE = _OFF[SHARED_LEVELS[0]]
SH_ROWS = _OFF[SHARED_LEVELS[-1] + 1] - SH_BASE
SH_ALLOC = (SH_ROWS + 7) // 8 * 8

HBM_LEVELS = list(range(SHARED_LEVELS[-1] + 1, N_LEVELS))
STREAM_LEVELS = SHARED_LEVELS + HBM_LEVELS

OUT_W = 2 * N_LEVELS


def _hash_mod(h_u32, hs):
    if hs & (hs - 1) == 0:
        return jnp.bitwise_and(h_u32, jnp.uint32(hs - 1))
    return jnp.remainder(h_u32, jnp.uint32(hs))


def _corner_setup(xv, b, res):
    x0 = xv[0, pl.ds(b, LANES)]
    x1 = xv[1, pl.ds(b, LANES)]
    x2 = xv[2, pl.ds(b, LANES)]
    r = jnp.float32(res)
    xs0, xs1, xs2 = x0 * r, x1 * r, x2 * r
    xi0 = xs0.astype(jnp.int32)
    xi1 = xs1.astype(jnp.int32)
    xi2 = xs2.astype(jnp.int32)
    xf0 = xs0 - xi0.astype(jnp.float32)
    xf1 = xs1 - xi1.astype(jnp.float32)
    xf2 = xs2 - xi2.astype(jnp.float32)
    a0 = xi0.astype(jnp.uint32)
    a1 = a0 + jnp.uint32(1)
    b0 = xi1.astype(jnp.uint32) * jnp.uint32(P1)
    b1 = b0 + jnp.uint32(P1)
    c0 = xi2.astype(jnp.uint32) * jnp.uint32(P2)
    c1 = c0 + jnp.uint32(P2)
    u = (jnp.float32(1.0) - xf0, xf0)
    v = (jnp.float32(1.0) - xf1, xf1)
    t = (jnp.float32(1.0) - xf2, xf2)
    return (a0, a1), (b0, b1), (c0, c1), u, v, t


def kernel(x, tables):
    x_t = x.T
    t0 = tables[:, 0]
    t1 = tables[:, 1]

    mesh = plsc.VectorSubcoreMesh(core_axis_name="c", subcore_axis_name="s")

    @functools.partial(
        pl.kernel,
        mesh=mesh,
        compiler_params=pltpu.CompilerParams(
            needs_layout_passes=False, use_tc_tiling_on_sc=False),
        out_type=jax.ShapeDtypeStruct((N_POINTS * OUT_W,), jnp.float32),
        scratch_types=[
            pltpu.VMEM((IN_DIM, CP), jnp.float32),
            pltpu.VMEM((STAGE_PAD,), jnp.float32),
            pltpu.VMEM((STAGE_PAD,), jnp.float32),
            pltpu.VMEM_SHARED((SH_ALLOC,), jnp.float32),
            pltpu.VMEM_SHARED((SH_ALLOC,), jnp.float32),
            pltpu.VMEM((8 * CP,), jnp.int32),
            pltpu.VMEM((2 * 8 * CP,), jnp.float32),
            pltpu.VMEM((8 * CP,), jnp.float32),
            pltpu.VMEM((CP * OUT_W,), jnp.float32),
            pltpu.SemaphoreType.DMA,
        ],
    )
    def grid_kernel(x_hbm, t0_hbm, t1_hbm, out_hbm, xv, ts0, ts1, sh0, sh1,
                    idxbuf, gbuf, wbuf, obuf, sem):
        wid = lax.axis_index("s") * NC + lax.axis_index("c")

        pltpu.sync_copy(t0_hbm.at[pl.ds(0, STAGE_PAD)], ts0)
        pltpu.sync_copy(t1_hbm.at[pl.ds(0, STAGE_PAD)], ts1)

        @pl.when(lax.axis_index("s") == 0)
        def _stage_shared():
            pltpu.sync_copy(t0_hbm.at[pl.ds(SH_BASE, SH_ALLOC)], sh0)
            pltpu.sync_copy(t1_hbm.at[pl.ds(SH_BASE, SH_ALLOC)], sh1)

        plsc.subcore_barrier()

        iota = lax.iota(jnp.int32, LANES)
        oidx = iota * OUT_W

        def out_store(b, l, acc0, acc1):
            sidx = oidx + (b * OUT_W + 2 * l)
            plsc.store_scatter(obuf, [sidx], acc0)
            plsc.store_scatter(obuf, [sidx + 1], acc1)

        def run_fused_level(l):
            res, hs, off = _RES[l], _HS[l], _OFF[l]

            def fgroup(g, c):
                b = g * LANES
                (a0, a1), (b0, b1), (c0, c1), u, v, t = \
                    _corner_setup(xv, b, res)
                acc0 = jnp.zeros((LANES,), jnp.float32)
                acc1 = jnp.zeros((LANES,), jnp.float32)
                for j in range(8):
                    j0, j1, j2 = j & 1, (j >> 1) & 1, (j >> 2) & 1
                    h = (a1 if j0 else a0) ^ (b1 if j1 else b0) \
                        ^ (c1 if j2 else c0)
                    rid = (_hash_mod(h, hs)).astype(jnp.int32) + off
                    w = (u[j0] * v[j1]) * t[j2]
                    g0 = plsc.load_gather(ts0, [rid])
                    g1 = plsc.load_gather(ts1, [rid])
                    acc0 = acc0 + g0 * w
                    acc1 = acc1 + g1 * w
                out_store(b, l, acc0, acc1)
                return c

            lax.fori_loop(0, NG, fgroup, 0)

        def run_stream_level(l):
            res, hs, off = _RES[l], _HS[l], _OFF[l]
            shared = l in SHARED_LEVELS
            row_off = (off - SH_BASE) if shared else off
            s0 = sh0 if shared else t0_hbm
            s1 = sh1 if shared else t1_hbm

            def igroup(g, c):
                b = g * LANES
                blk = g * GBLK
                (a0, a1), (b0, b1), (c0, c1), u, v, t = \
                    _corner_setup(xv, b, res)
                for j in range(8):
                    j0, j1, j2 = j & 1, (j >> 1) & 1, (j >> 2) & 1
                    h = (a1 if j0 else a0) ^ (b1 if j1 else b0) \
                        ^ (c1 if j2 else c0)
                    rid = (_hash_mod(h, hs)).astype(jnp.int32) + row_off
                    w = (u[j0] * v[j1]) * t[j2]
                    idxbuf[pl.ds(blk + j * LANES, LANES)] = rid
                    wbuf[pl.ds(blk + j * LANES, LANES)] = w
                pltpu.async_copy(s0.at[idxbuf.at[pl.ds(blk, GBLK)]],
                                 gbuf.at[pl.ds(blk, GBLK)], sem)
                pltpu.async_copy(s1.at[idxbuf.at[pl.ds(blk, GBLK)]],
                                 gbuf.at[pl.ds(8 * CP + blk, GBLK)], sem)
                return c

            lax.fori_loop(0, NG, igroup, 0)

            for g in range(NG):
                pltpu.make_async_copy(
                    t0_hbm.at[pl.ds(0, GBLK)],
                    gbuf.at[pl.ds(g * GBLK, GBLK)], sem).wait()
                pltpu.make_async_copy(
                    t0_hbm.at[pl.ds(0, GBLK)],
                    gbuf.at[pl.ds(8 * CP + g * GBLK, GBLK)], sem).wait()

            def rgroup(g, c):
                b = g * LANES
                blk = g * GBLK
                acc0 = jnp.zeros((LANES,), jnp.float32)
                acc1 = jnp.zeros((LANES,), jnp.float32)
                for j in range(8):
                    pos = blk + j * LANES
                    wv = wbuf[pl.ds(pos, LANES)]
                    g0 = gbuf[pl.ds(pos, LANES)]
                    g1 = gbuf[pl.ds(8 * CP + pos, LANES)]
                    acc0 = acc0 + g0 * wv
                    acc1 = acc1 + g1 * wv
                out_store(b, l, acc0, acc1)
                return c

            lax.fori_loop(0, NG, rgroup, 0)

        def chunk_body(ci, carry):
            base = wid * PTS_PER_W + ci * CP
            pltpu.sync_copy(x_hbm.at[:, pl.ds(base, CP)], xv)

            run_stream_level(STREAM_LEVELS[0])
            for l in range(N_TILE_STAGED):
                run_fused_level(l)
            for l in STREAM_LEVELS[1:]:
                run_stream_level(l)

            pltpu.sync_copy(obuf, out_hbm.at[pl.ds(base * OUT_W, CP * OUT_W)])
            return carry

        lax.fori_loop(0, N_CHUNKS, chunk_body, 0)

    out_flat = grid_kernel(x_t, t0, t1)
    return out_flat.reshape(N_POINTS, OUT_W)


# speedup vs baseline: 9.6264x; 1.0222x over previous
import functools
import math

import jax
import jax.numpy as jnp
from jax import lax
from jax.experimental import pallas as pl
from jax.experimental.pallas import tpu as pltpu
from jax.experimental.pallas import tpu_sc as plsc

IN_DIM = 3
N_LEVELS = 16
F = 2
LOG2_HASHMAP = 19
BASE_RES = 16
DESIRED_RES = 512
N_POINTS = 524288

P1 = 2654435761
P2 = 805459861

_beta = math.exp((math.log(DESIRED_RES) - math.log(BASE_RES)) / (BASE_RES - 1))
_RES = [int(math.floor(BASE_RES * _beta ** l)) for l in range(N_LEVELS)]
_HS = [min(r ** IN_DIM, 2 ** LOG2_HASHMAP) for r in _RES]
_OFF = [0]
for _h in _HS:
    _OFF.append(_OFF[-1] + _h)
TOTAL_ROWS = _OFF[-1]

NC = 2
NS = 16
NW = NC * NS
LANES = 16

PTS_PER_W = N_POINTS // NW
CP = 512
N_CHUNKS = PTS_PER_W // CP
NG = CP // LANES
GBLK = 8 * LANES

N_TILE_STAGED = 2
STAGE_ROWS = _OFF[N_TILE_STAGED]
STAGE_PAD = (STAGE_ROWS + 7) // 8 * 8

SHARED_LEVELS = [2, 3, 4, 5, 6]
SH_BASE = _OFF[SHARED_LEVELS[0]]
SH_ROWS = _OFF[SHARED_LEVELS[-1] + 1] - SH_BASE
SH_ALLOC = (SH_ROWS + 7) // 8 * 8

HBM_LEVELS = list(range(SHARED_LEVELS[-1] + 1, N_LEVELS))
STREAM_LEVELS = SHARED_LEVELS + HBM_LEVELS

OUT_W = 2 * N_LEVELS


def _hash_mod(h_u32, hs):
    if hs & (hs - 1) == 0:
        return jnp.bitwise_and(h_u32, jnp.uint32(hs - 1))
    return jnp.remainder(h_u32, jnp.uint32(hs))


def _corner_setup(xv, b, res):
    x0 = xv[0, pl.ds(b, LANES)]
    x1 = xv[1, pl.ds(b, LANES)]
    x2 = xv[2, pl.ds(b, LANES)]
    r = jnp.float32(res)
    xs0, xs1, xs2 = x0 * r, x1 * r, x2 * r
    xi0 = xs0.astype(jnp.int32)
    xi1 = xs1.astype(jnp.int32)
    xi2 = xs2.astype(jnp.int32)
    xf0 = xs0 - xi0.astype(jnp.float32)
    xf1 = xs1 - xi1.astype(jnp.float32)
    xf2 = xs2 - xi2.astype(jnp.float32)
    a0 = xi0.astype(jnp.uint32)
    a1 = a0 + jnp.uint32(1)
    b0 = xi1.astype(jnp.uint32) * jnp.uint32(P1)
    b1 = b0 + jnp.uint32(P1)
    c0 = xi2.astype(jnp.uint32) * jnp.uint32(P2)
    c1 = c0 + jnp.uint32(P2)
    u = (jnp.float32(1.0) - xf0, xf0)
    v = (jnp.float32(1.0) - xf1, xf1)
    t = (jnp.float32(1.0) - xf2, xf2)
    return (a0, a1), (b0, b1), (c0, c1), u, v, t


def kernel(x, tables):
    x_t = x.T
    t0 = tables[:, 0]
    t1 = tables[:, 1]

    mesh = plsc.VectorSubcoreMesh(core_axis_name="c", subcore_axis_name="s")

    @functools.partial(
        pl.kernel,
        mesh=mesh,
        compiler_params=pltpu.CompilerParams(
            needs_layout_passes=False, use_tc_tiling_on_sc=False),
        out_type=jax.ShapeDtypeStruct((N_POINTS * OUT_W,), jnp.float32),
        scratch_types=[
            pltpu.VMEM((IN_DIM, CP), jnp.float32),
            pltpu.VMEM((STAGE_PAD,), jnp.float32),
            pltpu.VMEM((STAGE_PAD,), jnp.float32),
            pltpu.VMEM_SHARED((SH_ALLOC,), jnp.float32),
            pltpu.VMEM_SHARED((SH_ALLOC,), jnp.float32),
            pltpu.VMEM((8 * CP,), jnp.int32),
            pltpu.VMEM((2 * 8 * CP,), jnp.float32),
            pltpu.VMEM((8 * CP,), jnp.float32),
            pltpu.VMEM((CP * OUT_W,), jnp.float32),
            pltpu.SemaphoreType.DMA,
        ],
    )
    def grid_kernel(x_hbm, t0_hbm, t1_hbm, out_hbm, xv, ts0, ts1, sh0, sh1,
                    idxbuf, gbuf, wbuf, obuf, sem):
        wid = lax.axis_index("s") * NC + lax.axis_index("c")

        pltpu.sync_copy(t0_hbm.at[pl.ds(0, STAGE_PAD)], ts0)
        pltpu.sync_copy(t1_hbm.at[pl.ds(0, STAGE_PAD)], ts1)

        @pl.when(lax.axis_index("s") == 0)
        def _stage_shared():
            pltpu.sync_copy(t0_hbm.at[pl.ds(SH_BASE, SH_ALLOC)], sh0)
            pltpu.sync_copy(t1_hbm.at[pl.ds(SH_BASE, SH_ALLOC)], sh1)

        plsc.subcore_barrier()

        iota = lax.iota(jnp.int32, LANES)
        oidx = iota * OUT_W

        def out_store(b, l, acc0, acc1):
            sidx = oidx + (b * OUT_W + 2 * l)
            plsc.store_scatter(obuf, [sidx], acc0)
            plsc.store_scatter(obuf, [sidx + 1], acc1)

        def run_fused_level(l):
            res, hs, off = _RES[l], _HS[l], _OFF[l]

            @plsc.parallel_loop(0, NG, unroll=2)
            def fgroup(g):
                b = g * LANES
                (a0, a1), (b0, b1), (c0, c1), u, v, t = \
                    _corner_setup(xv, b, res)
                acc0 = jnp.zeros((LANES,), jnp.float32)
                acc1 = jnp.zeros((LANES,), jnp.float32)
                for j in range(8):
                    j0, j1, j2 = j & 1, (j >> 1) & 1, (j >> 2) & 1
                    h = (a1 if j0 else a0) ^ (b1 if j1 else b0) \
                        ^ (c1 if j2 else c0)
                    rid = (_hash_mod(h, hs)).astype(jnp.int32) + off
                    w = (u[j0] * v[j1]) * t[j2]
                    g0 = plsc.load_gather(ts0, [rid])
                    g1 = plsc.load_gather(ts1, [rid])
                    acc0 = acc0 + g0 * w
                    acc1 = acc1 + g1 * w
                out_store(b, l, acc0, acc1)

        def run_stream_level(l):
            res, hs, off = _RES[l], _HS[l], _OFF[l]
            shared = l in SHARED_LEVELS
            row_off = (off - SH_BASE) if shared else off
            s0 = sh0 if shared else t0_hbm
            s1 = sh1 if shared else t1_hbm

            @plsc.parallel_loop(0, NG, unroll=2)
            def igroup(g):
                b = g * LANES
                blk = g * GBLK
                (a0, a1), (b0, b1), (c0, c1), u, v, t = \
                    _corner_setup(xv, b, res)
                for j in range(8):
                    j0, j1, j2 = j & 1, (j >> 1) & 1, (j >> 2) & 1
                    h = (a1 if j0 else a0) ^ (b1 if j1 else b0) \
                        ^ (c1 if j2 else c0)
                    rid = (_hash_mod(h, hs)).astype(jnp.int32) + row_off
                    w = (u[j0] * v[j1]) * t[j2]
                    idxbuf[pl.ds(blk + j * LANES, LANES)] = rid
                    wbuf[pl.ds(blk + j * LANES, LANES)] = w
                pltpu.async_copy(s0.at[idxbuf.at[pl.ds(blk, GBLK)]],
                                 gbuf.at[pl.ds(blk, GBLK)], sem)
                pltpu.async_copy(s1.at[idxbuf.at[pl.ds(blk, GBLK)]],
                                 gbuf.at[pl.ds(8 * CP + blk, GBLK)], sem)

            for g in range(NG):
                pltpu.make_async_copy(
                    t0_hbm.at[pl.ds(0, GBLK)],
                    gbuf.at[pl.ds(g * GBLK, GBLK)], sem).wait()
                pltpu.make_async_copy(
                    t0_hbm.at[pl.ds(0, GBLK)],
                    gbuf.at[pl.ds(8 * CP + g * GBLK, GBLK)], sem).wait()

            @plsc.parallel_loop(0, NG, unroll=2)
            def rgroup(g):
                b = g * LANES
                blk = g * GBLK
                acc0 = jnp.zeros((LANES,), jnp.float32)
                acc1 = jnp.zeros((LANES,), jnp.float32)
                for j in range(8):
                    pos = blk + j * LANES
                    wv = wbuf[pl.ds(pos, LANES)]
                    g0 = gbuf[pl.ds(pos, LANES)]
                    g1 = gbuf[pl.ds(8 * CP + pos, LANES)]
                    acc0 = acc0 + g0 * wv
                    acc1 = acc1 + g1 * wv
                out_store(b, l, acc0, acc1)

        def chunk_body(ci, carry):
            base = wid * PTS_PER_W + ci * CP
            pltpu.sync_copy(x_hbm.at[:, pl.ds(base, CP)], xv)

            run_stream_level(STREAM_LEVELS[0])
            for l in range(N_TILE_STAGED):
                run_fused_level(l)
            for l in STREAM_LEVELS[1:]:
                run_stream_level(l)

            pltpu.sync_copy(obuf, out_hbm.at[pl.ds(base * OUT_W, CP * OUT_W)])
            return carry

        lax.fori_loop(0, N_CHUNKS, chunk_body, 0)

    out_flat = grid_kernel(x_t, t0, t1)
    return out_flat.reshape(N_POINTS, OUT_W)
